# Initial kernel scaffold; baseline (speedup 1.0000x reference)
import jax
import jax.numpy as jnp
from jax.experimental import pallas as pl


def kernel(x, edge_index, W1l, W1r, b1, W2l, W2r, b2):
    raise NotImplementedError("write your pallas kernel here")


# speedup vs baseline: 8.9367x; 8.9367x over previous
import functools

import jax
import jax.numpy as jnp
from jax import lax
from jax.experimental import pallas as pl
from jax.experimental.pallas import tpu as pltpu
from jax.experimental.pallas import tpu_sc as plsc

N = 10000
NP = 10240
D = 128
H = 128
C = 40
E = 320000
CW = 48
NSC = 2
NTILE = 16
WORKERS = NSC * NTILE
G = 128
CH = 80
NPH = 2
CHP = CH // NPH
EPW = CH * G
EPAD = WORKERS * EPW
RPT = NP // NTILE


def _make_segsum(with_count):
  mesh = plsc.VectorSubcoreMesh(core_axis_name="c", subcore_axis_name="s")
  out_type = [jax.ShapeDtypeStruct((NSC, NP, H), jnp.float32)]
  scratch = [
      pltpu.VMEM((CHP, G), jnp.int32),
      pltpu.VMEM((CHP, G), jnp.int32),
      pltpu.VMEM((2, G, H), jnp.float32),
      pltpu.VMEM_SHARED((NP, H), jnp.float32),
      pltpu.SemaphoreType.DMA,
  ]
  if with_count:
    out_type.append(jax.ShapeDtypeStruct((NSC, NP), jnp.float32))
    scratch.append(pltpu.VMEM((G,), jnp.float32))
    scratch.append(pltpu.VMEM_SHARED((NP,), jnp.float32))

  @functools.partial(pl.kernel, out_type=out_type, mesh=mesh,
                     scratch_types=scratch)
  def seg(y_hbm, src_hbm, dst_hbm, z_hbm, zc_hbm, *refs):
    if with_count:
      out_hbm, outc_hbm, src_v, dst_v, rows_v, acc_sh, sem, ones_v, cnt_sh = refs
    else:
      out_hbm, src_v, dst_v, rows_v, acc_sh, sem = refs
    c = lax.axis_index("c")
    s = lax.axis_index("s")
    wid = c * NTILE + s

    pltpu.sync_copy(z_hbm, acc_sh.at[pl.ds(s * RPT, RPT)])
    if with_count:
      pltpu.sync_copy(zc_hbm, cnt_sh.at[pl.ds(s * RPT, RPT)])
      for i in range(G // 16):
        ones_v[pl.ds(i * 16, 16)] = jnp.ones((16,), jnp.float32)

    def g_start(jj, b):
      pltpu.make_async_copy(y_hbm.at[src_v.at[jj]], rows_v.at[b], sem).start()

    def g_wait(jj, b):
      pltpu.make_async_copy(y_hbm.at[src_v.at[jj]], rows_v.at[b], sem).wait()

    def body(t, carry):
      j0 = 2 * t
      j1 = j0 + 1
      g_wait(j0, 0)
      g_start(j1, 1)
      pltpu.sync_copy(rows_v.at[0], acc_sh.at[dst_v.at[j0]], add=True)
      if with_count:
        pltpu.sync_copy(ones_v, cnt_sh.at[dst_v.at[j0]], add=True)
      g_wait(j1, 1)

      @pl.when(t < CHP // 2 - 1)
      def _():
        g_start(j0 + 2, 0)

      pltpu.sync_copy(rows_v.at[1], acc_sh.at[dst_v.at[j1]], add=True)
      if with_count:
        pltpu.sync_copy(ones_v, cnt_sh.at[dst_v.at[j1]], add=True)
      return carry

    for ph in range(NPH):
      chunks = pl.ds(ph * CHP, CHP)
      pltpu.sync_copy(src_hbm.at[wid].at[chunks], src_v)
      pltpu.sync_copy(dst_hbm.at[wid].at[chunks], dst_v)
      g_start(0, 0)
      if ph == 0:
        plsc.subcore_barrier()
      lax.fori_loop(0, CHP // 2, body, 0)
    plsc.subcore_barrier()
    rows = pl.ds(s * RPT, RPT)
    pltpu.sync_copy(acc_sh.at[rows], out_hbm.at[c].at[rows])
    if with_count:
      pltpu.sync_copy(cnt_sh.at[rows], outc_hbm.at[c].at[rows])

  return seg


_segsum_cnt = _make_segsum(True)
_segsum = _make_segsum(False)


def _k1_body(x_ref, wl_ref, wr_ref, b_ref, y_ref, xr_ref):
  xb = x_ref[...]
  y_ref[...] = jnp.dot(xb, wl_ref[...], preferred_element_type=jnp.float32)
  xr_ref[...] = jnp.dot(xb, wr_ref[...],
                        preferred_element_type=jnp.float32) + b_ref[...]


def _k2_body(p_ref, cnt_ref, xr_ref, wr_ref, b_ref, h_ref, hr_ref):
  sums = p_ref[0] + p_ref[1]
  cnt = cnt_ref[0] + cnt_ref[1]
  mean = sums / jnp.maximum(cnt, 1.0)
  h = jnp.maximum(mean + xr_ref[...], 0.0)
  h_ref[...] = h
  hr_ref[...] = jnp.dot(h, wr_ref[...],
                        preferred_element_type=jnp.float32) + b_ref[...]


def _k3_body(p2_ref, cnt_ref, hr_ref, wl_ref, o_ref):
  sums = p2_ref[0] + p2_ref[1]
  cnt = cnt_ref[0] + cnt_ref[1]
  mean = sums / jnp.maximum(cnt, 1.0)
  o = jnp.dot(mean, wl_ref[...], preferred_element_type=jnp.float32)
  o_ref[...] = (o + hr_ref[...])[:, :C]


def kernel(x, edge_index, W1l, W1r, b1, W2l, W2r, b2):
  f32 = jnp.float32
  src = edge_index[0]
  dst = edge_index[1]
  pad = N + (jnp.arange(EPAD - E, dtype=jnp.int32) % (NP - N))
  src_p = jnp.concatenate([src, pad]).reshape(WORKERS, CH, G)
  dst_p = jnp.concatenate([dst, pad]).reshape(WORKERS, CH, G)

  w1lt = W1l.T
  w1rt = W1r.T
  b1r = b1.reshape(1, H)
  w2lt = jnp.concatenate([W2l.T, jnp.zeros((H, CW - C), f32)], axis=1)
  w2rt = jnp.concatenate([W2r.T, jnp.zeros((H, CW - C), f32)], axis=1)
  b2r = jnp.concatenate([b2, jnp.zeros((CW - C,), f32)]).reshape(1, CW)
  z1 = jnp.zeros((RPT, H), f32)
  zc = jnp.zeros((RPT,), f32)

  bm = 256
  y1, xr = pl.pallas_call(
      _k1_body,
      grid=(NP // bm,),
      in_specs=[
          pl.BlockSpec((bm, D), lambda i: (i, 0)),
          pl.BlockSpec((D, H), lambda i: (0, 0)),
          pl.BlockSpec((D, H), lambda i: (0, 0)),
          pl.BlockSpec((1, H), lambda i: (0, 0)),
      ],
      out_specs=[
          pl.BlockSpec((bm, H), lambda i: (i, 0)),
          pl.BlockSpec((bm, H), lambda i: (i, 0)),
      ],
      out_shape=[
          jax.ShapeDtypeStruct((NP, H), f32),
          jax.ShapeDtypeStruct((NP, H), f32),
      ],
  )(x, w1lt, w1rt, b1r)

  p1, cnt = _segsum_cnt(y1, src_p, dst_p, z1, zc)
  cnt = cnt.reshape(NSC, NP, 1)

  h, hr = pl.pallas_call(
      _k2_body,
      grid=(NP // bm,),
      in_specs=[
          pl.BlockSpec((NSC, bm, H), lambda i: (0, i, 0)),
          pl.BlockSpec((NSC, bm, 1), lambda i: (0, i, 0)),
          pl.BlockSpec((bm, H), lambda i: (i, 0)),
          pl.BlockSpec((H, CW), lambda i: (0, 0)),
          pl.BlockSpec((1, CW), lambda i: (0, 0)),
      ],
      out_specs=[
          pl.BlockSpec((bm, H), lambda i: (i, 0)),
          pl.BlockSpec((bm, CW), lambda i: (i, 0)),
      ],
      out_shape=[
          jax.ShapeDtypeStruct((NP, H), f32),
          jax.ShapeDtypeStruct((NP, CW), f32),
      ],
  )(p1, cnt, xr, w2rt, b2r)

  p2 = _segsum(h, src_p, dst_p, z1, zc)[0]

  bm3 = 400
  out = pl.pallas_call(
      _k3_body,
      grid=(N // bm3,),
      in_specs=[
          pl.BlockSpec((NSC, bm3, H), lambda i: (0, i, 0)),
          pl.BlockSpec((NSC, bm3, 1), lambda i: (0, i, 0)),
          pl.BlockSpec((bm3, CW), lambda i: (i, 0)),
          pl.BlockSpec((H, CW), lambda i: (0, 0)),
      ],
      out_specs=pl.BlockSpec((bm3, C), lambda i: (i, 0)),
      out_shape=jax.ShapeDtypeStruct((N, C), f32),
  )(p2, cnt, hr, w2lt)
  return out
